# baseline (speedup 1.0000x reference)
import functools

import jax
import jax.numpy as jnp
from jax import lax
from jax.experimental import pallas as pl
from jax.experimental.pallas import tpu as pltpu
from jax.experimental.pallas import tpu_sc as plsc

D_MODEL = 64
SCALE = 8.0
ROW = 128

_info = plsc.get_sparse_core_info()
_NC = _info.num_cores
_NS = _info.num_subcores
_NW = _NC * _NS
_C = 128


def _embed_sc(idxt2d, table_padded, n_seq, n_batch):
    n_chunks = idxt2d.shape[0]
    chunks_per_w = n_chunks // _NW
    cblocks = n_batch // _C

    mesh = plsc.VectorSubcoreMesh(core_axis_name="c", subcore_axis_name="s")

    @functools.partial(
        pl.kernel,
        mesh=mesh,
        compiler_params=pltpu.CompilerParams(needs_layout_passes=False),
        out_type=jax.ShapeDtypeStruct((n_seq, D_MODEL, n_batch), jnp.float32),
        scratch_types=[
            pltpu.VMEM((chunks_per_w, _C), jnp.int32),
            pltpu.VMEM((_C, ROW), jnp.float32),
            pltpu.VMEM((_C, ROW), jnp.float32),
            pltpu.VMEM((D_MODEL, _C), jnp.float32),
            pltpu.VMEM((D_MODEL, _C), jnp.float32),
            pltpu.SemaphoreType.DMA,
            pltpu.SemaphoreType.DMA,
            pltpu.SemaphoreType.DMA,
            pltpu.SemaphoreType.DMA,
        ],
    )
    def k(idx_hbm, table_hbm, out_hbm, idx_v, gbuf0, gbuf1, tbuf0, tbuf1,
          g0, g1, o0, o1):
        wid = lax.axis_index("s") * _NC + lax.axis_index("c")
        kbase = wid * chunks_per_w
        pltpu.sync_copy(idx_hbm.at[pl.ds(kbase, chunks_per_w)], idx_v)

        gbufs = (gbuf0, gbuf1)
        tbufs = (tbuf0, tbuf1)
        gsems = (g0, g1)
        osems = (o0, o1)
        iota16 = lax.iota(jnp.int32, 16)

        def gather(l, b):
            pltpu.async_copy(table_hbm.at[idx_v.at[l]], gbufs[b], gsems[b])

        def wait_gather(b):
            pltpu.make_async_copy(table_hbm.at[idx_v.at[0]], gbufs[b],
                                  gsems[b]).wait()

        def transpose_scale(b):
            gbuf, tbuf = gbufs[b], tbufs[b]

            def fstep(f4, _):
                for df in range(4):
                    f = f4 * 4 + df
                    lidx = jnp.broadcast_to(f, (16,)).astype(jnp.int32)
                    for g in range(8):
                        tvec = iota16 + (16 * g)
                        v = plsc.load_gather(gbuf, [tvec, lidx])
                        tbuf[f, pl.ds(16 * g, 16)] = v * SCALE
                return ()

            lax.fori_loop(0, D_MODEL // 4, fstep, ())

        def put(l, b):
            kk = kbase + l
            j = kk // cblocks
            c = kk % cblocks
            pltpu.async_copy(tbufs[b],
                             out_hbm.at[j, :, pl.ds(c * _C, _C)],
                             osems[b])

        def wait_put(b):
            pltpu.make_async_copy(tbufs[b],
                                  out_hbm.at[0, :, pl.ds(0, _C)],
                                  osems[b]).wait()

        gather(0, 0)
        wait_gather(0)
        gather(1, 1)
        transpose_scale(0)
        put(0, 0)
        wait_gather(1)
        gather(2, 0)
        transpose_scale(1)
        put(1, 1)

        def step(ll, _):
            l = 2 + 2 * ll
            wait_gather(0)
            gather(l + 1, 1)
            wait_put(0)
            transpose_scale(0)
            put(l, 0)
            wait_gather(1)
            gather(l + 2, 0)
            wait_put(1)
            transpose_scale(1)
            put(l + 1, 1)
            return ()

        lax.fori_loop(0, (chunks_per_w - 4) // 2, step, ())

        wait_gather(0)
        gather(chunks_per_w - 1, 1)
        wait_put(0)
        transpose_scale(0)
        put(chunks_per_w - 2, 0)
        wait_gather(1)
        wait_put(1)
        transpose_scale(1)
        put(chunks_per_w - 1, 1)
        wait_put(0)
        wait_put(1)

    return k(idxt2d, table_padded)


def kernel(token_ids, embedding_weights):
    s0, s1 = token_ids.shape
    idxt = token_ids.T.astype(jnp.int32).reshape(-1, _C)
    tab = jnp.pad(embedding_weights, ((0, 0), (0, ROW - D_MODEL)))
    out = _embed_sc(idxt, tab, s1, s0)
    return jnp.transpose(out, (2, 0, 1))


# speedup vs baseline: 1.6609x; 1.6609x over previous
import functools

import jax
import jax.numpy as jnp
from jax import lax
from jax.experimental import pallas as pl
from jax.experimental.pallas import tpu as pltpu
from jax.experimental.pallas import tpu_sc as plsc

D_MODEL = 64
SCALE = 8.0
ROW = 128

_info = plsc.get_sparse_core_info()
_NC = _info.num_cores
_NS = _info.num_subcores
_NW = _NC * _NS
_C = 128


def _embed_sc(idxt2d, table_padded, n_seq, n_batch):
    n_chunks = idxt2d.shape[0]
    chunks_per_w = n_chunks // _NW
    cblocks = n_batch // _C

    mesh = plsc.VectorSubcoreMesh(core_axis_name="c", subcore_axis_name="s")

    @functools.partial(
        pl.kernel,
        mesh=mesh,
        compiler_params=pltpu.CompilerParams(needs_layout_passes=False),
        out_type=jax.ShapeDtypeStruct((n_seq, D_MODEL, n_batch), jnp.float32),
        scratch_types=[
            pltpu.VMEM((chunks_per_w, _C), jnp.int32),
            pltpu.VMEM((_C, ROW), jnp.float32),
            pltpu.VMEM((_C, ROW), jnp.float32),
            pltpu.VMEM((D_MODEL, _C), jnp.float32),
            pltpu.VMEM((D_MODEL, _C), jnp.float32),
            pltpu.SemaphoreType.DMA,
            pltpu.SemaphoreType.DMA,
            pltpu.SemaphoreType.DMA,
            pltpu.SemaphoreType.DMA,
        ],
    )
    def k(idx_hbm, table_hbm, out_hbm, idx_v, gbuf0, gbuf1, tbuf0, tbuf1,
          g0, g1, o0, o1):
        wid = lax.axis_index("s") * _NC + lax.axis_index("c")
        kbase = wid * chunks_per_w
        pltpu.sync_copy(idx_hbm.at[pl.ds(kbase, chunks_per_w)], idx_v)

        gbufs = (gbuf0, gbuf1)
        tbufs = (tbuf0, tbuf1)
        gsems = (g0, g1)
        osems = (o0, o1)
        iota16 = lax.iota(jnp.int32, 16)

        def gather(l, b):
            pltpu.async_copy(table_hbm.at[idx_v.at[l]], gbufs[b], gsems[b])

        def wait_gather(b):
            pltpu.make_async_copy(table_hbm.at[idx_v.at[0]], gbufs[b],
                                  gsems[b]).wait()

        def transpose_scale(b):
            gbuf, tbuf = gbufs[b], tbufs[b]

            def gstep(g, _):
                tvec = iota16 + 16 * g
                rots = [((iota16 + r) & 15) for r in range(16)]
                for fb in range(D_MODEL // 16):
                    for r in range(16):
                        lvec = rots[r] + 16 * fb
                        v = plsc.load_gather(gbuf, [tvec, lvec])
                        plsc.store_scatter(tbuf, [lvec, tvec], v * SCALE)
                return ()

            lax.fori_loop(0, _C // 16, gstep, ())

        def put(l, b):
            kk = kbase + l
            j = kk // cblocks
            c = kk % cblocks
            pltpu.async_copy(tbufs[b],
                             out_hbm.at[j, :, pl.ds(c * _C, _C)],
                             osems[b])

        def wait_put(b):
            pltpu.make_async_copy(tbufs[b],
                                  out_hbm.at[0, :, pl.ds(0, _C)],
                                  osems[b]).wait()

        gather(0, 0)
        wait_gather(0)
        gather(1, 1)
        transpose_scale(0)
        put(0, 0)
        wait_gather(1)
        gather(2, 0)
        transpose_scale(1)
        put(1, 1)

        def step(ll, _):
            l = 2 + 2 * ll
            wait_gather(0)
            gather(l + 1, 1)
            wait_put(0)
            transpose_scale(0)
            put(l, 0)
            wait_gather(1)
            gather(l + 2, 0)
            wait_put(1)
            transpose_scale(1)
            put(l + 1, 1)
            return ()

        lax.fori_loop(0, (chunks_per_w - 4) // 2, step, ())

        wait_gather(0)
        gather(chunks_per_w - 1, 1)
        wait_put(0)
        transpose_scale(0)
        put(chunks_per_w - 2, 0)
        wait_gather(1)
        wait_put(1)
        transpose_scale(1)
        put(chunks_per_w - 1, 1)
        wait_put(0)
        wait_put(1)

    return k(idxt2d, table_padded)


def kernel(token_ids, embedding_weights):
    s0, s1 = token_ids.shape
    idxt = token_ids.T.astype(jnp.int32).reshape(-1, _C)
    tab = jnp.pad(embedding_weights, ((0, 0), (0, ROW - D_MODEL)))
    out = _embed_sc(idxt, tab, s1, s0)
    return jnp.transpose(out, (2, 0, 1))
